# baseline (speedup 1.0000x reference)
import functools

import jax
import jax.numpy as jnp
from jax import lax
from jax.experimental import pallas as pl
from jax.experimental.pallas import tpu as pltpu
from jax.experimental.pallas import tpu_sc as plsc


def _topk_kernel(x1_ref, x2_ref, idx_ref, w_ref, *, bn, S):
    b = pl.program_id(0)
    x1 = x1_ref[0]
    x2 = x2_ref[0]

    dot2 = lax.dot_general(
        (-2.0 * x1).astype(jnp.bfloat16), x2.astype(jnp.bfloat16),
        (((1,), (0,)), ((), ())),
        preferred_element_type=jnp.float32)
    x1sq = jnp.sum(x1 * x1, axis=1, keepdims=True)
    x2sq = ((x2[0:1] * x2[0:1] + x2[1:2] * x2[1:2])
            + x2[2:3] * x2[2:3])
    dists = (dot2 + x1sq) + x2sq

    col = lax.broadcasted_iota(jnp.int32, (bn, S), 1).astype(jnp.float32)
    big = jnp.float32(3.0e38)
    s_f = jnp.float32(S)
    d = dists
    ms = []
    hits = []
    for k in range(3):
        m = jnp.min(d, axis=1, keepdims=True)
        amin = jnp.min(jnp.where(d == m, col, s_f), axis=1, keepdims=True)
        hit = (col == amin).astype(jnp.float32)
        ms.append(m)
        hits.append(hit)
        if k < 2:
            d = jnp.where(hit > 0, big, d)
    r0 = 1.0 / (ms[0] + 1e-8)
    r1 = 1.0 / (ms[1] + 1e-8)
    r2 = 1.0 / (ms[2] + 1e-8)
    norm = r0 + r1 + r2

    col_row = lax.broadcasted_iota(jnp.int32, (1, S), 1).astype(jnp.float32)
    idx_rows = []
    for k in range(3):
        amin_lane = lax.dot_general(
            col_row, hits[k], (((1,), (1,)), ((), ())),
            precision=lax.Precision.HIGHEST,
            preferred_element_type=jnp.float32)
        idx_rows.append(amin_lane)

    base = (b * S).astype(jnp.float32)
    idx_ref[...] = (jnp.concatenate(idx_rows, axis=0)
                    + base + 0.5).astype(jnp.int32)
    for k in range(3):
        w_ref[k] = jnp.broadcast_to([r0, r1, r2][k] / norm, (bn, 16))


def _make_sc_interp(P, D, C):
    info = plsc.get_sparse_core_info()
    nc, ns = info.num_cores, info.num_subcores
    nw = nc * ns
    ppw = P // nw
    nchunks = ppw // C
    mesh = plsc.VectorSubcoreMesh(core_axis_name="c", subcore_axis_name="s")

    @functools.partial(
        pl.kernel, mesh=mesh,
        out_type=jax.ShapeDtypeStruct((P, D), jnp.float32),
        scratch_types=[
            pltpu.VMEM((3, C), jnp.int32),
            pltpu.VMEM((3, C, 16), jnp.float32),
            pltpu.VMEM((C, D), jnp.float32),
            pltpu.VMEM((C, D), jnp.float32),
            pltpu.VMEM((C, D), jnp.float32),
            pltpu.VMEM((C, D), jnp.float32),
            pltpu.SemaphoreType.DMA,
        ],
    )
    def sc_interp(table_hbm, idx_hbm, w_hbm, out_hbm,
                  idx_v, w_v, r0_v, r1_v, r2_v, out_v, sem):
        wid = lax.axis_index("s") * nc + lax.axis_index("c")

        def chunk_body(c, carry):
            base = wid * ppw + c * C
            for k in range(3):
                pltpu.sync_copy(idx_hbm.at[k, pl.ds(base, C)], idx_v.at[k])
                pltpu.sync_copy(w_hbm.at[k, pl.ds(base, C)], w_v.at[k])
            cp0 = pltpu.async_copy(table_hbm.at[idx_v.at[0]], r0_v, sem)
            cp1 = pltpu.async_copy(table_hbm.at[idx_v.at[1]], r1_v, sem)
            cp2 = pltpu.async_copy(table_hbm.at[idx_v.at[2]], r2_v, sem)
            cp0.wait()
            cp1.wait()
            cp2.wait()

            def pbody(p, pcarry):
                w0 = w_v[0, p, :]
                w1 = w_v[1, p, :]
                w2 = w_v[2, p, :]
                for dchunk in range(D // 16):
                    sl = pl.ds(dchunk * 16, 16)
                    out_v[p, sl] = ((w0 * r0_v[p, sl] + w1 * r1_v[p, sl])
                                    + w2 * r2_v[p, sl])
                return pcarry

            lax.fori_loop(0, C, pbody, 0)
            pltpu.sync_copy(out_v, out_hbm.at[pl.ds(base, C)])
            return carry

        lax.fori_loop(0, nchunks, chunk_body, 0)

    return sc_interp


def _mlp_kernel(p1_ref, it_ref, w1a_ref, w1b_ref, b1_ref,
                w2_ref, b2_ref, w3_ref, b3_ref, out_ref):
    y = lax.dot_general(
        w1a_ref[...], p1_ref[0], (((1,), (0,)), ((), ())),
        preferred_element_type=jnp.float32)
    y = y + lax.dot_general(
        w1b_ref[...], it_ref[...].astype(jnp.bfloat16), (((1,), (1,)), ((), ())),
        preferred_element_type=jnp.float32)
    h = jax.nn.relu(y + b1_ref[...])
    h = jax.nn.relu(
        lax.dot_general(w2_ref[...], h.astype(jnp.bfloat16),
                        (((1,), (0,)), ((), ())),
                        preferred_element_type=jnp.float32)
        + b2_ref[...])
    out_ref[0] = jax.nn.relu(
        lax.dot_general(w3_ref[...], h.astype(jnp.bfloat16),
                        (((1,), (0,)), ((), ())),
                        preferred_element_type=jnp.float32)
        + b3_ref[...])


def kernel(xyz1, xyz2, points1, points2, W1, b1, W2, b2, W3, b3):
    B, C, N = xyz1.shape
    S = xyz2.shape[2]
    D = points1.shape[1]
    O = W3.shape[0]
    P = B * N
    bn = 512
    grid = (B, N // bn)

    x1t = jnp.transpose(xyz1, (0, 2, 1))
    x1t = jnp.concatenate(
        [x1t, jnp.zeros((B, N, 8 - C), dtype=xyz1.dtype)], axis=2)
    x2p = jnp.concatenate(
        [xyz2, jnp.zeros((B, 8 - C, S), dtype=xyz2.dtype)], axis=1)

    ka = functools.partial(_topk_kernel, bn=bn, S=S)
    idxg, wts = pl.pallas_call(
        ka,
        grid=grid,
        in_specs=[
            pl.BlockSpec((1, bn, 8), lambda b, n: (b, n, 0)),
            pl.BlockSpec((1, 8, S), lambda b, n: (b, 0, 0)),
        ],
        out_specs=[
            pl.BlockSpec((3, bn), lambda b, n: (0, b * (N // bn) + n)),
            pl.BlockSpec((3, bn, 16), lambda b, n: (0, b * (N // bn) + n, 0)),
        ],
        out_shape=[
            jax.ShapeDtypeStruct((3, P), jnp.int32),
            jax.ShapeDtypeStruct((3, P, 16), jnp.float32),
        ],
    )(x1t, x2p)

    table = jnp.transpose(points2, (0, 2, 1)).reshape(B * S, D)
    interp = _make_sc_interp(P, D, 64)(table, idxg, wts)

    p1b = points1.astype(jnp.bfloat16)
    w1a = W1[:, :D].astype(jnp.bfloat16)
    w1b = W1[:, D:].astype(jnp.bfloat16)
    w2b = W2.astype(jnp.bfloat16)
    w3b = W3.astype(jnp.bfloat16)
    b1c = b1.reshape(-1, 1)
    b2c = b2.reshape(-1, 1)
    b3c = b3.reshape(-1, 1)

    return pl.pallas_call(
        _mlp_kernel,
        grid=grid,
        in_specs=[
            pl.BlockSpec((1, D, bn), lambda b, n: (b, 0, n)),
            pl.BlockSpec((bn, D), lambda b, n: (b * (N // bn) + n, 0)),
            pl.BlockSpec(w1a.shape, lambda b, n: (0, 0)),
            pl.BlockSpec(w1b.shape, lambda b, n: (0, 0)),
            pl.BlockSpec(b1c.shape, lambda b, n: (0, 0)),
            pl.BlockSpec(w2b.shape, lambda b, n: (0, 0)),
            pl.BlockSpec(b2c.shape, lambda b, n: (0, 0)),
            pl.BlockSpec(w3b.shape, lambda b, n: (0, 0)),
            pl.BlockSpec(b3c.shape, lambda b, n: (0, 0)),
        ],
        out_specs=pl.BlockSpec((1, O, bn), lambda b, n: (b, 0, n)),
        out_shape=jax.ShapeDtypeStruct((B, O, N), jnp.float32),
    )(p1b, interp, w1a, w1b, b1c, w2b, b2c, w3b, b3c)


# speedup vs baseline: 1.8117x; 1.8117x over previous
import functools

import jax
import jax.numpy as jnp
from jax import lax
from jax.experimental import pallas as pl
from jax.experimental.pallas import tpu as pltpu
from jax.experimental.pallas import tpu_sc as plsc


def _topk_kernel(x1_ref, x2_ref, idx_ref, w_ref, *, bn, S):
    b = pl.program_id(0)
    x1t = x1_ref[0]
    x2t = x2_ref[0]

    dot2 = lax.dot_general(
        (-2.0 * x2t).astype(jnp.bfloat16), x1t.astype(jnp.bfloat16),
        (((1,), (0,)), ((), ())),
        preferred_element_type=jnp.float32)
    x1sq = ((x1t[0:1] * x1t[0:1] + x1t[1:2] * x1t[1:2])
            + x1t[2:3] * x1t[2:3])
    x2sq = ((x2t[:, 0:1] * x2t[:, 0:1] + x2t[:, 1:2] * x2t[:, 1:2])
            + x2t[:, 2:3] * x2t[:, 2:3])
    dists = (dot2 + x1sq) + x2sq

    row = lax.broadcasted_iota(jnp.int32, (S, bn), 0).astype(jnp.float32)
    big = jnp.float32(3.0e38)
    s_f = jnp.float32(S)
    d = dists
    ms = []
    amins = []
    for k in range(3):
        m = jnp.min(d, axis=0, keepdims=True)
        amin = jnp.min(jnp.where(d == m, row, s_f), axis=0, keepdims=True)
        ms.append(m)
        amins.append(amin)
        if k < 2:
            d = jnp.where(row == amin, big, d)
    r0 = 1.0 / (ms[0] + 1e-8)
    r1 = 1.0 / (ms[1] + 1e-8)
    r2 = 1.0 / (ms[2] + 1e-8)
    norm = r0 + r1 + r2

    base = (b * S).astype(jnp.float32)
    idx_ref[...] = (jnp.concatenate(amins, axis=0)
                    + base + 0.5).astype(jnp.int32)

    for k in range(3):
        w_lane = [r0, r1, r2][k] / norm
        w_ref[k] = jnp.broadcast_to(w_lane.reshape(bn, 1), (bn, 16))


def _make_sc_interp(P, D, C):
    info = plsc.get_sparse_core_info()
    nc, ns = info.num_cores, info.num_subcores
    nw = nc * ns
    ppw = P // nw
    nchunks = ppw // C
    mesh = plsc.VectorSubcoreMesh(core_axis_name="c", subcore_axis_name="s")

    @functools.partial(
        pl.kernel, mesh=mesh,
        out_type=jax.ShapeDtypeStruct((P, D), jnp.float32),
        scratch_types=[
            pltpu.VMEM((3, C), jnp.int32),
            pltpu.VMEM((3, C, 16), jnp.float32),
            pltpu.VMEM((C, D), jnp.float32),
            pltpu.VMEM((C, D), jnp.float32),
            pltpu.VMEM((C, D), jnp.float32),
            pltpu.VMEM((C, D), jnp.float32),
            pltpu.SemaphoreType.DMA,
        ],
    )
    def sc_interp(table_hbm, idx_hbm, w_hbm, out_hbm,
                  idx_v, w_v, r0_v, r1_v, r2_v, out_v, sem):
        wid = lax.axis_index("s") * nc + lax.axis_index("c")

        def chunk_body(c, carry):
            base = wid * ppw + c * C
            for k in range(3):
                pltpu.sync_copy(idx_hbm.at[k, pl.ds(base, C)], idx_v.at[k])
                pltpu.sync_copy(w_hbm.at[k, pl.ds(base, C)], w_v.at[k])
            cp0 = pltpu.async_copy(table_hbm.at[idx_v.at[0]], r0_v, sem)
            cp1 = pltpu.async_copy(table_hbm.at[idx_v.at[1]], r1_v, sem)
            cp2 = pltpu.async_copy(table_hbm.at[idx_v.at[2]], r2_v, sem)
            cp0.wait()
            cp1.wait()
            cp2.wait()

            def pbody(p, pcarry):
                w0 = w_v[0, p, :]
                w1 = w_v[1, p, :]
                w2 = w_v[2, p, :]
                for dchunk in range(D // 16):
                    sl = pl.ds(dchunk * 16, 16)
                    out_v[p, sl] = ((w0 * r0_v[p, sl] + w1 * r1_v[p, sl])
                                    + w2 * r2_v[p, sl])
                return pcarry

            lax.fori_loop(0, C, pbody, 0)
            pltpu.sync_copy(out_v, out_hbm.at[pl.ds(base, C)])
            return carry

        lax.fori_loop(0, nchunks, chunk_body, 0)

    return sc_interp


def _mlp_kernel(p1_ref, it_ref, w1a_ref, w1b_ref, b1_ref,
                w2_ref, b2_ref, w3_ref, b3_ref, out_ref):
    y = lax.dot_general(
        w1a_ref[...], p1_ref[0], (((1,), (0,)), ((), ())),
        preferred_element_type=jnp.float32)
    y = y + lax.dot_general(
        w1b_ref[...], it_ref[...].astype(jnp.bfloat16), (((1,), (1,)), ((), ())),
        preferred_element_type=jnp.float32)
    h = jax.nn.relu(y + b1_ref[...])
    h = jax.nn.relu(
        lax.dot_general(w2_ref[...], h.astype(jnp.bfloat16),
                        (((1,), (0,)), ((), ())),
                        preferred_element_type=jnp.float32)
        + b2_ref[...])
    out_ref[0] = jax.nn.relu(
        lax.dot_general(w3_ref[...], h.astype(jnp.bfloat16),
                        (((1,), (0,)), ((), ())),
                        preferred_element_type=jnp.float32)
        + b3_ref[...])


def kernel(xyz1, xyz2, points1, points2, W1, b1, W2, b2, W3, b3):
    B, C, N = xyz1.shape
    S = xyz2.shape[2]
    D = points1.shape[1]
    O = W3.shape[0]
    P = B * N
    bn = 512
    grid = (B, N // bn)

    x1p = jnp.concatenate(
        [xyz1, jnp.zeros((B, 8 - C, N), dtype=xyz1.dtype)], axis=1)
    x2t = jnp.transpose(xyz2, (0, 2, 1))
    x2t = jnp.concatenate(
        [x2t, jnp.zeros((B, S, 8 - C), dtype=xyz2.dtype)], axis=2)

    ka = functools.partial(_topk_kernel, bn=bn, S=S)
    idxg, wts = pl.pallas_call(
        ka,
        grid=grid,
        in_specs=[
            pl.BlockSpec((1, 8, bn), lambda b, n: (b, 0, n)),
            pl.BlockSpec((1, S, 8), lambda b, n: (b, 0, 0)),
        ],
        out_specs=[
            pl.BlockSpec((3, bn), lambda b, n: (0, b * (N // bn) + n)),
            pl.BlockSpec((3, bn, 16), lambda b, n: (0, b * (N // bn) + n, 0)),
        ],
        out_shape=[
            jax.ShapeDtypeStruct((3, P), jnp.int32),
            jax.ShapeDtypeStruct((3, P, 16), jnp.float32),
        ],
    )(x1p, x2t)

    table = jnp.transpose(points2, (0, 2, 1)).reshape(B * S, D)
    interp = _make_sc_interp(P, D, 64)(table, idxg, wts)

    p1b = points1.astype(jnp.bfloat16)
    w1a = W1[:, :D].astype(jnp.bfloat16)
    w1b = W1[:, D:].astype(jnp.bfloat16)
    w2b = W2.astype(jnp.bfloat16)
    w3b = W3.astype(jnp.bfloat16)
    b1c = b1.reshape(-1, 1)
    b2c = b2.reshape(-1, 1)
    b3c = b3.reshape(-1, 1)

    return pl.pallas_call(
        _mlp_kernel,
        grid=grid,
        in_specs=[
            pl.BlockSpec((1, D, bn), lambda b, n: (b, 0, n)),
            pl.BlockSpec((bn, D), lambda b, n: (b * (N // bn) + n, 0)),
            pl.BlockSpec(w1a.shape, lambda b, n: (0, 0)),
            pl.BlockSpec(w1b.shape, lambda b, n: (0, 0)),
            pl.BlockSpec(b1c.shape, lambda b, n: (0, 0)),
            pl.BlockSpec(w2b.shape, lambda b, n: (0, 0)),
            pl.BlockSpec(b2c.shape, lambda b, n: (0, 0)),
            pl.BlockSpec(w3b.shape, lambda b, n: (0, 0)),
            pl.BlockSpec(b3c.shape, lambda b, n: (0, 0)),
        ],
        out_specs=pl.BlockSpec((1, O, bn), lambda b, n: (b, 0, n)),
        out_shape=jax.ShapeDtypeStruct((B, O, N), jnp.float32),
    )(p1b, interp, w1a, w1b, b1c, w2b, b2c, w3b, b3c)


# speedup vs baseline: 2.2208x; 1.2258x over previous
import functools

import jax
import jax.numpy as jnp
from jax import lax
from jax.experimental import pallas as pl
from jax.experimental.pallas import tpu as pltpu
from jax.experimental.pallas import tpu_sc as plsc


def _topk_kernel(x1_ref, x2_ref, idx_ref, w_ref, *, bn, S, b0):
    b = pl.program_id(0)
    x1t = x1_ref[0]
    x2t = x2_ref[0]

    dot2 = lax.dot_general(
        (-2.0 * x2t).astype(jnp.bfloat16), x1t.astype(jnp.bfloat16),
        (((1,), (0,)), ((), ())),
        preferred_element_type=jnp.float32)
    x1sq = ((x1t[0:1] * x1t[0:1] + x1t[1:2] * x1t[1:2])
            + x1t[2:3] * x1t[2:3])
    x2sq = ((x2t[:, 0:1] * x2t[:, 0:1] + x2t[:, 1:2] * x2t[:, 1:2])
            + x2t[:, 2:3] * x2t[:, 2:3])
    dists = (dot2 + x1sq) + x2sq

    row = lax.broadcasted_iota(jnp.int32, (S, bn), 0).astype(jnp.float32)
    big = jnp.float32(3.0e38)
    s_f = jnp.float32(S)
    d = dists
    ms = []
    amins = []
    for k in range(3):
        m = jnp.min(d, axis=0, keepdims=True)
        amin = jnp.min(jnp.where(d == m, row, s_f), axis=0, keepdims=True)
        ms.append(m)
        amins.append(amin)
        if k < 2:
            d = jnp.where(row == amin, big, d)
    r0 = 1.0 / (ms[0] + 1e-8)
    r1 = 1.0 / (ms[1] + 1e-8)
    r2 = 1.0 / (ms[2] + 1e-8)
    norm = r0 + r1 + r2

    base = ((b + b0) * S).astype(jnp.float32)
    idx_ref[...] = (jnp.concatenate(amins, axis=0)
                    + base + 0.5).astype(jnp.int32)

    for k in range(3):
        w_lane = [r0, r1, r2][k] / norm
        w_ref[k] = jnp.broadcast_to(w_lane.reshape(bn, 1), (bn, 16))


def _make_sc_interp(P, D, C):
    info = plsc.get_sparse_core_info()
    nc, ns = info.num_cores, info.num_subcores
    nw = nc * ns
    ppw = P // nw
    nchunks = ppw // C
    mesh = plsc.VectorSubcoreMesh(core_axis_name="c", subcore_axis_name="s")

    @functools.partial(
        pl.kernel, mesh=mesh,
        out_type=jax.ShapeDtypeStruct((P, D), jnp.float32),
        scratch_types=[
            pltpu.VMEM((3, C), jnp.int32),
            pltpu.VMEM((3, C, 16), jnp.float32),
            pltpu.VMEM((C, D), jnp.float32),
            pltpu.VMEM((C, D), jnp.float32),
            pltpu.VMEM((C, D), jnp.float32),
            pltpu.VMEM((C, D), jnp.float32),
            pltpu.SemaphoreType.DMA,
        ],
    )
    def sc_interp(table_hbm, idx_hbm, w_hbm, out_hbm,
                  idx_v, w_v, r0_v, r1_v, r2_v, out_v, sem):
        wid = lax.axis_index("s") * nc + lax.axis_index("c")

        def chunk_body(c, carry):
            base = wid * ppw + c * C
            for k in range(3):
                pltpu.sync_copy(idx_hbm.at[k, pl.ds(base, C)], idx_v.at[k])
                pltpu.sync_copy(w_hbm.at[k, pl.ds(base, C)], w_v.at[k])
            cp0 = pltpu.async_copy(table_hbm.at[idx_v.at[0]], r0_v, sem)
            cp1 = pltpu.async_copy(table_hbm.at[idx_v.at[1]], r1_v, sem)
            cp2 = pltpu.async_copy(table_hbm.at[idx_v.at[2]], r2_v, sem)
            cp0.wait()
            cp1.wait()
            cp2.wait()

            def pbody(p, pcarry):
                w0 = w_v[0, p, :]
                w1 = w_v[1, p, :]
                w2 = w_v[2, p, :]
                for dchunk in range(D // 16):
                    sl = pl.ds(dchunk * 16, 16)
                    out_v[p, sl] = ((w0 * r0_v[p, sl] + w1 * r1_v[p, sl])
                                    + w2 * r2_v[p, sl])
                return pcarry

            lax.fori_loop(0, C, pbody, 0)
            pltpu.sync_copy(out_v, out_hbm.at[pl.ds(base, C)])
            return carry

        lax.fori_loop(0, nchunks, chunk_body, 0)

    return sc_interp


def _mlp_kernel(p1_ref, it_ref, w1a_ref, w1b_ref, b1_ref,
                w2_ref, b2_ref, w3_ref, b3_ref, out_ref):
    y = lax.dot_general(
        w1a_ref[...], p1_ref[0], (((1,), (0,)), ((), ())),
        preferred_element_type=jnp.float32)
    y = y + lax.dot_general(
        w1b_ref[...], it_ref[...].astype(jnp.bfloat16), (((1,), (1,)), ((), ())),
        preferred_element_type=jnp.float32)
    h = jax.nn.relu(y + b1_ref[...])
    h = jax.nn.relu(
        lax.dot_general(w2_ref[...], h.astype(jnp.bfloat16),
                        (((1,), (0,)), ((), ())),
                        preferred_element_type=jnp.float32)
        + b2_ref[...])
    out_ref[0] = jax.nn.relu(
        lax.dot_general(w3_ref[...], h.astype(jnp.bfloat16),
                        (((1,), (0,)), ((), ())),
                        preferred_element_type=jnp.float32)
        + b3_ref[...])


def kernel(xyz1, xyz2, points1, points2, W1, b1, W2, b2, W3, b3):
    B, C, N = xyz1.shape
    S = xyz2.shape[2]
    D = points1.shape[1]
    O = W3.shape[0]
    P = B * N
    bn = 512
    grid = (B, N // bn)

    x1p = jnp.concatenate(
        [xyz1, jnp.zeros((B, 8 - C, N), dtype=xyz1.dtype)], axis=1)
    x2t = jnp.transpose(xyz2, (0, 2, 1))
    x2t = jnp.concatenate(
        [x2t, jnp.zeros((B, S, 8 - C), dtype=xyz2.dtype)], axis=2)

    table = jnp.transpose(points2, (0, 2, 1)).reshape(B * S, D)
    p1b = points1.astype(jnp.bfloat16)
    w1a = W1[:, :D].astype(jnp.bfloat16)
    w1b = W1[:, D:].astype(jnp.bfloat16)
    w2b = W2.astype(jnp.bfloat16)
    w3b = W3.astype(jnp.bfloat16)
    b1c = b1.reshape(-1, 1)
    b2c = b2.reshape(-1, 1)
    b3c = b3.reshape(-1, 1)

    nh = 2
    Bh = B // nh
    Ph = Bh * N
    sc_interp = _make_sc_interp(Ph, D, 64)
    grid = (Bh, N // bn)
    outs = []
    for h in range(nh):
        b0 = h * Bh

        ka = functools.partial(_topk_kernel, bn=bn, S=S, b0=b0)
        idxg, wts = pl.pallas_call(
            ka,
            grid=grid,
            in_specs=[
                pl.BlockSpec((1, 8, bn), lambda b, n: (b + b0, 0, n)),
                pl.BlockSpec((1, S, 8), lambda b, n: (b + b0, 0, 0)),
            ],
            out_specs=[
                pl.BlockSpec((3, bn), lambda b, n: (0, b * (N // bn) + n)),
                pl.BlockSpec((3, bn, 16),
                             lambda b, n: (0, b * (N // bn) + n, 0)),
            ],
            out_shape=[
                jax.ShapeDtypeStruct((3, Ph), jnp.int32),
                jax.ShapeDtypeStruct((3, Ph, 16), jnp.float32),
            ],
        )(x1p, x2t)

        interp = sc_interp(table, idxg, wts)

        outs.append(pl.pallas_call(
            _mlp_kernel,
            grid=grid,
            in_specs=[
                pl.BlockSpec((1, D, bn), lambda b, n: (b + b0, 0, n)),
                pl.BlockSpec((bn, D), lambda b, n: (b * (N // bn) + n, 0)),
                pl.BlockSpec(w1a.shape, lambda b, n: (0, 0)),
                pl.BlockSpec(w1b.shape, lambda b, n: (0, 0)),
                pl.BlockSpec(b1c.shape, lambda b, n: (0, 0)),
                pl.BlockSpec(w2b.shape, lambda b, n: (0, 0)),
                pl.BlockSpec(b2c.shape, lambda b, n: (0, 0)),
                pl.BlockSpec(w3b.shape, lambda b, n: (0, 0)),
                pl.BlockSpec(b3c.shape, lambda b, n: (0, 0)),
            ],
            out_specs=pl.BlockSpec((1, O, bn), lambda b, n: (b, 0, n)),
            out_shape=jax.ShapeDtypeStruct((Bh, O, N), jnp.float32),
        )(p1b, interp, w1a, w1b, b1c, w2b, b2c, w3b, b3c))

    return jnp.concatenate(outs, axis=0)
